# baseline (speedup 1.0000x reference)
import functools

import jax
import jax.numpy as jnp
from jax import lax
from jax.experimental import pallas as pl
from jax.experimental.pallas import tpu as pltpu
from jax.experimental.pallas import tpu_sc as plsc

DIM = 64
RANK = 16
E = 8
H = 512
W = 512

ROWS_A = 32
ROWS_C = 32
NA = H // ROWS_A
NC = H // ROWS_C
PC = ROWS_C * W

_PREC = jax.lax.Precision.HIGHEST
_NEG = -1e30


def _gelu_exact(v):
    return 0.5 * v * (1.0 + jax.lax.erf(v * 0.7071067811865476))


def _stats_kernel(x_ref, w1_ref, b1_ref, w2_ref, b2_ref, wg_ref, wf_ref,
                  p0_ref, p1_ref, p2_ref,
                  logits_ref, pw_ref, t_carry, x_carry, sums):
    i = pl.program_id(0)
    xc = x_ref[0]
    T = ROWS_A

    col = jax.lax.broadcasted_iota(jnp.int32, (DIM, T, W), 2)
    left = jnp.where(col == 0, 0.0, jnp.roll(xc, 1, axis=2))
    right = jnp.where(col == W - 1, 0.0, jnp.roll(xc, -1, axis=2))
    t = xc + left + right

    @pl.when(i == 0)
    def _init():
        t_carry[...] = jnp.zeros_like(t_carry)
        x_carry[...] = jnp.zeros_like(x_carry)
        sums[...] = jnp.zeros_like(sums)

    ext_t = jnp.concatenate([t_carry[...], t], axis=1)
    box = ext_t[:, 0:T] + ext_t[:, 1:T + 1] + ext_t[:, 2:T + 2]
    ext_x = jnp.concatenate([x_carry[...], xc[:, :T - 1]], axis=1)
    hp = 9.0 * ext_x - box
    g = _gelu_exact(hp)
    row = jax.lax.broadcasted_iota(jnp.int32, (DIM, T, W), 1)
    g = jnp.where((i == 0) & (row == 0), 0.0, g)
    sums[0, :] += jnp.sum(g, axis=(1, 2))
    sums[1, :] += jnp.sum(xc, axis=(1, 2))

    t_carry[...] = t[:, T - 2:T]
    x_carry[...] = xc[:, T - 1:T]

    @pl.when(i == NA - 1)
    def _finish():
        box_l = t[:, T - 2] + t[:, T - 1]
        hp_l = 9.0 * xc[:, T - 1] - box_l
        sums[0, :] += jnp.sum(_gelu_exact(hp_l), axis=1)
        inv = 1.0 / (H * W)
        f0 = sums[0:1, :] * inv
        pooled = sums[1:2, :] * inv
        h1 = _gelu_exact(
            lax.dot_general(f0, w1_ref[...], (((1,), (1,)), ((), ())),
                            precision=_PREC,
                            preferred_element_type=jnp.float32)
            + b1_ref[...])
        f = lax.dot_general(h1, w2_ref[...], (((1,), (1,)), ((), ())),
                            precision=_PREC,
                            preferred_element_type=jnp.float32) + b2_ref[...]
        logits = (
            lax.dot_general(pooled, wg_ref[...], (((1,), (1,)), ((), ())),
                            precision=_PREC,
                            preferred_element_type=jnp.float32)
            + lax.dot_general(f, wf_ref[...], (((1,), (1,)), ((), ())),
                              precision=_PREC,
                              preferred_element_type=jnp.float32))
        logits_ref[...] = jnp.concatenate(
            [logits, jnp.full((1, 16 - E), _NEG, jnp.float32)], axis=1)
        for k in range(RANK):
            pw_ref[0:E, pl.ds(DIM * k, DIM)] = p0_ref[:, k, :]
            pw_ref[E:2 * E, pl.ds(DIM * k, DIM)] = p1_ref[:, k, :]
        for c in range(DIM):
            pw_ref[2 * E:3 * E, pl.ds(RANK * c, RANK)] = p2_ref[:, c, :]


def _route_body(logits_hbm, pw_hbm,
                a_out, b_out, c_out, v_out,
                lg_v, prob_v, psort_v, idx_v, buf_v, pk_scr, c_scr, v_scr,
                sem):
    wid = lax.axis_index("s") * 2 + lax.axis_index("c")

    @pl.when(wid < 4)
    def _active():
        pltpu.sync_copy(logits_hbm, lg_v)
        l = lg_v[0, :]
        iota = jax.lax.iota(jnp.int32, 16)
        _, sv = plsc.sort_key_val(l, iota, descending=True)

        def _bfly_max(vec):
            for shift in (1, 2, 4, 8):
                prob_v[...] = vec
                vec = jnp.maximum(
                    vec, plsc.load_gather(prob_v,
                                          [jnp.bitwise_xor(iota, shift)]))
            return vec

        m16 = _bfly_max(l)
        e = jnp.exp(l - m16)
        s = e
        for shift in (1, 2, 4, 8):
            prob_v[...] = s
            s = s + plsc.load_gather(prob_v, [jnp.bitwise_xor(iota, shift)])
        p = e / s
        prob_v[...] = p
        psort_v[...] = plsc.load_gather(prob_v, [sv])

        def _gather(row_offset):
            idx_v[...] = sv + row_offset
            pltpu.async_copy(pw_hbm.at[idx_v.at[pl.ds(0, 8)]], buf_v,
                             sem).wait()

        def _pack_ab(out_ref):
            for j in range(2):
                for r in range(RANK):
                    for k in range(DIM // 16):
                        pk_scr[RANK * j + r, pl.ds(16 * k, 16)] = (
                            buf_v[j, pl.ds(DIM * r + 16 * k, 16)])
            pltpu.sync_copy(pk_scr, out_ref)

        @pl.when(wid == 0)
        def _a():
            _gather(0)
            _pack_ab(a_out)
            v_scr[0, :] = psort_v[...]
            pltpu.sync_copy(v_scr, v_out)

        @pl.when(wid == 1)
        def _b():
            _gather(E)
            _pack_ab(b_out)

        def _pack_c(j):
            _gather(2 * E)
            vj = _bfly_max(jnp.where(iota == j, psort_v[...], -3.4e38))
            for c in range(DIM):
                c_scr[c, :] = buf_v[j, pl.ds(RANK * c, RANK)] * vj
            pltpu.sync_copy(c_scr, c_out.at[j])

        @pl.when(wid == 2)
        def _c0():
            _pack_c(0)

        @pl.when(wid == 3)
        def _c1():
            _pack_c(1)


def _apply_kernel(x_ref, s_ref, a_ref, b_ref, c_ref, wout_ref, v_ref, o_ref,
                  m_scr, wx_scr):
    @pl.when(pl.program_id(0) == 0)
    def _fold():
        wout = wout_ref[...]
        m_scr[0] = jnp.dot(wout, c_ref[0], preferred_element_type=jnp.float32,
                           precision=_PREC)
        m_scr[1] = jnp.dot(wout, c_ref[1], preferred_element_type=jnp.float32,
                           precision=_PREC)
        v = v_ref[...]
        col = jax.lax.broadcasted_iota(jnp.int32, (1, 16), 1)
        s = jnp.sum(jnp.where(col < 2, v, 0.0))
        wx_scr[...] = s * wout

    def _dg(m, t):
        return lax.dot_general(m, t, (((1,), (0,)), ((), ())),
                               precision=_PREC,
                               preferred_element_type=jnp.float32)

    X = x_ref[0]
    S = s_ref[0]
    a = _dg(a_ref[...], X)
    b = _dg(b_ref[...], S)
    y = a * (b * jax.nn.sigmoid(b))
    o_ref[0] = (_dg(m_scr[0], y[:RANK]) + _dg(m_scr[1], y[RANK:])
                + _dg(wx_scr[...], X))


def _route(logits16, pw):
    return pl.kernel(
        _route_body,
        [
            jax.ShapeDtypeStruct((2 * RANK, DIM), jnp.float32),
            jax.ShapeDtypeStruct((2 * RANK, DIM), jnp.float32),
            jax.ShapeDtypeStruct((2, DIM, RANK), jnp.float32),
            jax.ShapeDtypeStruct((1, 16), jnp.float32),
        ],
        mesh=plsc.VectorSubcoreMesh(core_axis_name="c", subcore_axis_name="s"),
        compiler_params=pltpu.CompilerParams(needs_layout_passes=False),
        scratch_types=[
            pltpu.VMEM((1, 16), jnp.float32),
            pltpu.VMEM((16,), jnp.float32),
            pltpu.VMEM((16,), jnp.float32),
            pltpu.VMEM((16,), jnp.int32),
            pltpu.VMEM((E, RANK * DIM), jnp.float32),
            pltpu.VMEM((2 * RANK, DIM), jnp.float32),
            pltpu.VMEM((DIM, RANK), jnp.float32),
            pltpu.VMEM((1, 16), jnp.float32),
            pltpu.SemaphoreType.DMA,
        ],
    )(logits16, pw)


@jax.jit
def kernel(x, shared, W_mlp1, b_mlp1, W_mlp2, b_mlp2, Wg, Wf, P0, P1, P2, Wout):
    full = lambda s: pl.BlockSpec(s, lambda i: tuple(0 for _ in s))
    logits16, pw = pl.pallas_call(
        _stats_kernel,
        grid=(NA,),
        in_specs=[
            pl.BlockSpec((1, DIM, ROWS_A, W), lambda i: (0, 0, i, 0)),
            full((2 * DIM, DIM)), full((1, 2 * DIM)),
            full((DIM, 2 * DIM)), full((1, DIM)),
            full((E, DIM)), full((E, DIM)),
            full((E, RANK, DIM)), full((E, RANK, DIM)), full((E, DIM, RANK)),
        ],
        out_specs=[
            pl.BlockSpec((1, 16), lambda i: (0, 0)),
            pl.BlockSpec((3 * E, RANK * DIM), lambda i: (0, 0)),
        ],
        out_shape=[
            jax.ShapeDtypeStruct((1, 16), jnp.float32),
            jax.ShapeDtypeStruct((3 * E, RANK * DIM), jnp.float32),
        ],
        scratch_shapes=[
            pltpu.VMEM((DIM, 2, W), jnp.float32),
            pltpu.VMEM((DIM, 1, W), jnp.float32),
            pltpu.VMEM((2, DIM), jnp.float32),
        ],
    )(x, W_mlp1, b_mlp1.reshape(1, -1), W_mlp2, b_mlp2.reshape(1, -1),
      Wg, Wf, P0, P1, P2)

    a2, b2, c2, vs = _route(logits16, pw)

    out = pl.pallas_call(
        _apply_kernel,
        grid=(NC,),
        in_specs=[
            pl.BlockSpec((1, DIM, ROWS_C, W), lambda i: (0, 0, i, 0)),
            pl.BlockSpec((1, DIM, ROWS_C, W), lambda i: (0, 0, i, 0)),
            full((2 * RANK, DIM)), full((2 * RANK, DIM)),
            full((2, DIM, RANK)), full((DIM, DIM)), full((1, 16)),
        ],
        out_specs=pl.BlockSpec((1, DIM, ROWS_C, W), lambda i: (0, 0, i, 0)),
        out_shape=jax.ShapeDtypeStruct((1, DIM, H, W), jnp.float32),
        scratch_shapes=[
            pltpu.VMEM((2, DIM, RANK), jnp.float32),
            pltpu.VMEM((DIM, DIM), jnp.float32),
        ],
        compiler_params=pltpu.CompilerParams(
            vmem_limit_bytes=100 * 1024 * 1024),
    )(x, shared, a2, b2, c2, Wout, vs)
    return out


# speedup vs baseline: 1.1058x; 1.1058x over previous
import functools

import jax
import jax.numpy as jnp
from jax import lax
from jax.experimental import pallas as pl
from jax.experimental.pallas import tpu as pltpu
from jax.experimental.pallas import tpu_sc as plsc

DIM = 64
RANK = 16
E = 8
H = 512
W = 512

ROWS_A = 32
ROWS_C = 32
NA = H // ROWS_A
NC = H // ROWS_C
PC = ROWS_C * W

_PREC = jax.lax.Precision.HIGHEST
_NEG = -1e30


def _gelu_exact(v):
    return 0.5 * v * (1.0 + jax.lax.erf(v * 0.7071067811865476))


def _stats_kernel(x_ref, w1_ref, b1_ref, w2_ref, b2_ref, wg_ref, wf_ref,
                  p0_ref, p1_ref, p2_ref,
                  logits_ref, pw_ref, t_carry, x_carry, sums):
    i = pl.program_id(0)
    xc = x_ref[0]
    T = ROWS_A

    col = jax.lax.broadcasted_iota(jnp.int32, (DIM, T, W), 2)
    left = jnp.where(col == 0, 0.0, jnp.roll(xc, 1, axis=2))
    right = jnp.where(col == W - 1, 0.0, jnp.roll(xc, -1, axis=2))
    t = xc + left + right

    @pl.when(i == 0)
    def _init():
        t_carry[...] = jnp.zeros_like(t_carry)
        x_carry[...] = jnp.zeros_like(x_carry)
        sums[...] = jnp.zeros_like(sums)

    ext_t = jnp.concatenate([t_carry[...], t], axis=1)
    box = ext_t[:, 0:T] + ext_t[:, 1:T + 1] + ext_t[:, 2:T + 2]
    ext_x = jnp.concatenate([x_carry[...], xc[:, :T - 1]], axis=1)
    hp = 9.0 * ext_x - box
    g = _gelu_exact(hp)
    row = jax.lax.broadcasted_iota(jnp.int32, (DIM, T, W), 1)
    g = jnp.where((i == 0) & (row == 0), 0.0, g)
    sums[0, :] += jnp.sum(g, axis=(1, 2))
    sums[1, :] += jnp.sum(xc, axis=(1, 2))

    t_carry[...] = t[:, T - 2:T]
    x_carry[...] = xc[:, T - 1:T]

    @pl.when(i == NA - 1)
    def _finish():
        box_l = t[:, T - 2] + t[:, T - 1]
        hp_l = 9.0 * xc[:, T - 1] - box_l
        sums[0, :] += jnp.sum(_gelu_exact(hp_l), axis=1)
        inv = 1.0 / (H * W)
        f0 = sums[0:1, :] * inv
        pooled = sums[1:2, :] * inv
        h1 = _gelu_exact(
            lax.dot_general(f0, w1_ref[...], (((1,), (1,)), ((), ())),
                            precision=_PREC,
                            preferred_element_type=jnp.float32)
            + b1_ref[...])
        f = lax.dot_general(h1, w2_ref[...], (((1,), (1,)), ((), ())),
                            precision=_PREC,
                            preferred_element_type=jnp.float32) + b2_ref[...]
        logits = (
            lax.dot_general(pooled, wg_ref[...], (((1,), (1,)), ((), ())),
                            precision=_PREC,
                            preferred_element_type=jnp.float32)
            + lax.dot_general(f, wf_ref[...], (((1,), (1,)), ((), ())),
                              precision=_PREC,
                              preferred_element_type=jnp.float32))
        logits_ref[...] = jnp.concatenate(
            [logits, jnp.full((1, 16 - E), _NEG, jnp.float32)], axis=1)
        for k in range(RANK):
            pw_ref[0:E, pl.ds(DIM * k, DIM)] = p0_ref[:, k, :]
            pw_ref[E:2 * E, pl.ds(DIM * k, DIM)] = p1_ref[:, k, :]
        for c in range(DIM):
            pw_ref[2 * E:3 * E, pl.ds(RANK * c, RANK)] = p2_ref[:, c, :]


def _route_body(logits_hbm, pw_hbm,
                a_out, b_out, c_out, v_out,
                lg_v, prob_v, psort_v, idx_v, buf_v, pk_scr, c_scr, v_scr,
                sem):
    wid = lax.axis_index("s") * 2 + lax.axis_index("c")

    @pl.when(wid < 4)
    def _active():
        pltpu.sync_copy(logits_hbm, lg_v)
        l = lg_v[0, :]
        iota = jax.lax.iota(jnp.int32, 16)
        _, sv = plsc.sort_key_val(l, iota, descending=True)

        def _bfly_max(vec):
            for shift in (1, 2, 4, 8):
                prob_v[...] = vec
                vec = jnp.maximum(
                    vec, plsc.load_gather(prob_v,
                                          [jnp.bitwise_xor(iota, shift)]))
            return vec

        m16 = _bfly_max(l)
        e = jnp.exp(l - m16)
        s = e
        for shift in (1, 2, 4, 8):
            prob_v[...] = s
            s = s + plsc.load_gather(prob_v, [jnp.bitwise_xor(iota, shift)])
        p = e / s
        prob_v[...] = p
        psort_v[...] = plsc.load_gather(prob_v, [sv])

        def _gather(row_offset):
            idx_v[...] = sv + row_offset
            pltpu.async_copy(pw_hbm.at[idx_v.at[pl.ds(0, 8)]], buf_v,
                             sem).wait()

        def _pack_ab(out_ref):
            for j in range(2):
                for r in range(RANK):
                    for k in range(DIM // 16):
                        pk_scr[RANK * j + r, pl.ds(16 * k, 16)] = (
                            buf_v[j, pl.ds(DIM * r + 16 * k, 16)])
            pltpu.sync_copy(pk_scr, out_ref)

        @pl.when(wid == 0)
        def _a():
            _gather(0)
            _pack_ab(a_out)
            v_scr[0, :] = psort_v[...]
            pltpu.sync_copy(v_scr, v_out)

        @pl.when(wid == 1)
        def _b():
            _gather(E)
            _pack_ab(b_out)

        def _pack_c(j):
            _gather(2 * E)
            vj = _bfly_max(jnp.where(iota == j, psort_v[...], -3.4e38))
            for c in range(DIM):
                c_scr[c, :] = buf_v[j, pl.ds(RANK * c, RANK)] * vj
            pltpu.sync_copy(c_scr, c_out.at[j])

        @pl.when(wid == 2)
        def _c0():
            _pack_c(0)

        @pl.when(wid == 3)
        def _c1():
            _pack_c(1)


def _apply_kernel(x_ref, s_ref, a_ref, b_ref, c_ref, wout_ref, v_ref, o_ref,
                  m_scr, wx_scr):
    @pl.when(pl.program_id(0) == 0)
    def _fold():
        wout = wout_ref[...]
        m_scr[0] = jnp.dot(wout, c_ref[0], preferred_element_type=jnp.float32,
                           precision=_PREC)
        m_scr[1] = jnp.dot(wout, c_ref[1], preferred_element_type=jnp.float32,
                           precision=_PREC)
        v = v_ref[...]
        col = jax.lax.broadcasted_iota(jnp.int32, (1, 16), 1)
        s = jnp.sum(jnp.where(col < 2, v, 0.0))
        wx_scr[...] = s * wout

    X = jnp.concatenate([x_ref[0, :, r, :] for r in range(ROWS_C)], axis=1)
    S = jnp.concatenate([s_ref[0, :, r, :] for r in range(ROWS_C)], axis=1)
    a = jnp.dot(a_ref[...], X, preferred_element_type=jnp.float32,
                precision=_PREC)
    b = jnp.dot(b_ref[...], S, preferred_element_type=jnp.float32,
                precision=_PREC)
    y = a * (b * jax.nn.sigmoid(b))
    o2 = (jnp.dot(m_scr[0], y[:RANK], preferred_element_type=jnp.float32,
                  precision=_PREC)
          + jnp.dot(m_scr[1], y[RANK:], preferred_element_type=jnp.float32,
                    precision=_PREC)
          + jnp.dot(wx_scr[...], X, preferred_element_type=jnp.float32,
                    precision=_PREC))
    for r in range(ROWS_C):
        o_ref[0, :, r, :] = o2[:, W * r:W * (r + 1)]


def _route(logits16, pw):
    return pl.kernel(
        _route_body,
        [
            jax.ShapeDtypeStruct((2 * RANK, DIM), jnp.float32),
            jax.ShapeDtypeStruct((2 * RANK, DIM), jnp.float32),
            jax.ShapeDtypeStruct((2, DIM, RANK), jnp.float32),
            jax.ShapeDtypeStruct((1, 16), jnp.float32),
        ],
        mesh=plsc.VectorSubcoreMesh(core_axis_name="c", subcore_axis_name="s"),
        compiler_params=pltpu.CompilerParams(needs_layout_passes=False),
        scratch_types=[
            pltpu.VMEM((1, 16), jnp.float32),
            pltpu.VMEM((16,), jnp.float32),
            pltpu.VMEM((16,), jnp.float32),
            pltpu.VMEM((16,), jnp.int32),
            pltpu.VMEM((E, RANK * DIM), jnp.float32),
            pltpu.VMEM((2 * RANK, DIM), jnp.float32),
            pltpu.VMEM((DIM, RANK), jnp.float32),
            pltpu.VMEM((1, 16), jnp.float32),
            pltpu.SemaphoreType.DMA,
        ],
    )(logits16, pw)


@jax.jit
def kernel(x, shared, W_mlp1, b_mlp1, W_mlp2, b_mlp2, Wg, Wf, P0, P1, P2, Wout):
    full = lambda s: pl.BlockSpec(s, lambda i: tuple(0 for _ in s))
    logits16, pw = pl.pallas_call(
        _stats_kernel,
        grid=(NA,),
        in_specs=[
            pl.BlockSpec((1, DIM, ROWS_A, W), lambda i: (0, 0, i, 0)),
            full((2 * DIM, DIM)), full((1, 2 * DIM)),
            full((DIM, 2 * DIM)), full((1, DIM)),
            full((E, DIM)), full((E, DIM)),
            full((E, RANK, DIM)), full((E, RANK, DIM)), full((E, DIM, RANK)),
        ],
        out_specs=[
            pl.BlockSpec((1, 16), lambda i: (0, 0)),
            pl.BlockSpec((3 * E, RANK * DIM), lambda i: (0, 0)),
        ],
        out_shape=[
            jax.ShapeDtypeStruct((1, 16), jnp.float32),
            jax.ShapeDtypeStruct((3 * E, RANK * DIM), jnp.float32),
        ],
        scratch_shapes=[
            pltpu.VMEM((DIM, 2, W), jnp.float32),
            pltpu.VMEM((DIM, 1, W), jnp.float32),
            pltpu.VMEM((2, DIM), jnp.float32),
        ],
    )(x, W_mlp1, b_mlp1.reshape(1, -1), W_mlp2, b_mlp2.reshape(1, -1),
      Wg, Wf, P0, P1, P2)

    a2, b2, c2, vs = _route(logits16, pw)

    out = pl.pallas_call(
        _apply_kernel,
        grid=(NC,),
        in_specs=[
            pl.BlockSpec((1, DIM, ROWS_C, W), lambda i: (0, 0, i, 0)),
            pl.BlockSpec((1, DIM, ROWS_C, W), lambda i: (0, 0, i, 0)),
            full((2 * RANK, DIM)), full((2 * RANK, DIM)),
            full((2, DIM, RANK)), full((DIM, DIM)), full((1, 16)),
        ],
        out_specs=pl.BlockSpec((1, DIM, ROWS_C, W), lambda i: (0, 0, i, 0)),
        out_shape=jax.ShapeDtypeStruct((1, DIM, H, W), jnp.float32),
        scratch_shapes=[
            pltpu.VMEM((2, DIM, RANK), jnp.float32),
            pltpu.VMEM((DIM, DIM), jnp.float32),
        ],
        compiler_params=pltpu.CompilerParams(
            vmem_limit_bytes=100 * 1024 * 1024),
    )(x, shared, a2, b2, c2, Wout, vs)
    return out


# speedup vs baseline: 1.9139x; 1.7307x over previous
import functools

import jax
import jax.numpy as jnp
from jax import lax
from jax.experimental import pallas as pl
from jax.experimental.pallas import tpu as pltpu
from jax.experimental.pallas import tpu_sc as plsc

DIM = 64
RANK = 16
E = 8
H = 512
W = 512

ROWS_A = 32
ROWS_C = 64
NA = H // ROWS_A
NC = H // ROWS_C
PC = ROWS_C * W

_PREC = jax.lax.Precision.HIGHEST
_APREC = jax.lax.Precision.DEFAULT
_NEG = -1e30


def _gelu_exact(v):
    return 0.5 * v * (1.0 + jax.lax.erf(v * 0.7071067811865476))


def _stats_kernel(x_ref, w1_ref, b1_ref, w2_ref, b2_ref, wg_ref, wf_ref,
                  p0_ref, p1_ref, p2_ref,
                  logits_ref, pw_ref, t_carry, x_carry, sums):
    i = pl.program_id(0)
    xc = x_ref[0]
    T = ROWS_A

    col = jax.lax.broadcasted_iota(jnp.int32, (DIM, T, W), 2)
    left = jnp.where(col == 0, 0.0, jnp.roll(xc, 1, axis=2))
    right = jnp.where(col == W - 1, 0.0, jnp.roll(xc, -1, axis=2))
    t = xc + left + right

    @pl.when(i == 0)
    def _init():
        t_carry[...] = jnp.zeros_like(t_carry)
        x_carry[...] = jnp.zeros_like(x_carry)
        sums[...] = jnp.zeros_like(sums)

    ext_t = jnp.concatenate([t_carry[...], t], axis=1)
    box = ext_t[:, 0:T] + ext_t[:, 1:T + 1] + ext_t[:, 2:T + 2]
    ext_x = jnp.concatenate([x_carry[...], xc[:, :T - 1]], axis=1)
    hp = 9.0 * ext_x - box
    g = _gelu_exact(hp)
    row = jax.lax.broadcasted_iota(jnp.int32, (DIM, T, W), 1)
    g = jnp.where((i == 0) & (row == 0), 0.0, g)
    sums[0, :] += jnp.sum(g, axis=(1, 2))
    sums[1, :] += jnp.sum(xc, axis=(1, 2))

    t_carry[...] = t[:, T - 2:T]
    x_carry[...] = xc[:, T - 1:T]

    @pl.when(i == NA - 1)
    def _finish():
        box_l = t[:, T - 2] + t[:, T - 1]
        hp_l = 9.0 * xc[:, T - 1] - box_l
        sums[0, :] += jnp.sum(_gelu_exact(hp_l), axis=1)
        inv = 1.0 / (H * W)
        f0 = sums[0:1, :] * inv
        pooled = sums[1:2, :] * inv
        h1 = _gelu_exact(
            lax.dot_general(f0, w1_ref[...], (((1,), (1,)), ((), ())),
                            precision=_PREC,
                            preferred_element_type=jnp.float32)
            + b1_ref[...])
        f = lax.dot_general(h1, w2_ref[...], (((1,), (1,)), ((), ())),
                            precision=_PREC,
                            preferred_element_type=jnp.float32) + b2_ref[...]
        logits = (
            lax.dot_general(pooled, wg_ref[...], (((1,), (1,)), ((), ())),
                            precision=_PREC,
                            preferred_element_type=jnp.float32)
            + lax.dot_general(f, wf_ref[...], (((1,), (1,)), ((), ())),
                              precision=_PREC,
                              preferred_element_type=jnp.float32))
        logits_ref[...] = jnp.concatenate(
            [logits, jnp.full((1, 16 - E), _NEG, jnp.float32)], axis=1)
        for k in range(RANK):
            pw_ref[0:E, pl.ds(DIM * k, DIM)] = p0_ref[:, k, :]
            pw_ref[E:2 * E, pl.ds(DIM * k, DIM)] = p1_ref[:, k, :]
        for c in range(DIM):
            pw_ref[2 * E:3 * E, pl.ds(RANK * c, RANK)] = p2_ref[:, c, :]


def _route_body(logits_hbm, pw_hbm,
                a_out, b_out, c_out, v_out,
                lg_v, prob_v, psort_v, idx_v, buf_v, pk_scr, c_scr, v_scr,
                sem):
    wid = lax.axis_index("s") * 2 + lax.axis_index("c")

    @pl.when(wid < 4)
    def _active():
        pltpu.sync_copy(logits_hbm, lg_v)
        l = lg_v[0, :]
        iota = jax.lax.iota(jnp.int32, 16)
        _, sv = plsc.sort_key_val(l, iota, descending=True)

        def _bfly_max(vec):
            for shift in (1, 2, 4, 8):
                prob_v[...] = vec
                vec = jnp.maximum(
                    vec, plsc.load_gather(prob_v,
                                          [jnp.bitwise_xor(iota, shift)]))
            return vec

        m16 = _bfly_max(l)
        e = jnp.exp(l - m16)
        s = e
        for shift in (1, 2, 4, 8):
            prob_v[...] = s
            s = s + plsc.load_gather(prob_v, [jnp.bitwise_xor(iota, shift)])
        p = e / s
        prob_v[...] = p
        psort_v[...] = plsc.load_gather(prob_v, [sv])

        def _gather(row_offset):
            idx_v[...] = sv + row_offset
            pltpu.async_copy(pw_hbm.at[idx_v.at[pl.ds(0, 8)]], buf_v,
                             sem).wait()

        def _pack_ab(out_ref):
            for j in range(2):
                for r in range(RANK):
                    for k in range(DIM // 16):
                        pk_scr[RANK * j + r, pl.ds(16 * k, 16)] = (
                            buf_v[j, pl.ds(DIM * r + 16 * k, 16)])
            pltpu.sync_copy(pk_scr, out_ref)

        @pl.when(wid == 0)
        def _a():
            _gather(0)
            _pack_ab(a_out)
            v_scr[0, :] = psort_v[...]
            pltpu.sync_copy(v_scr, v_out)

        @pl.when(wid == 1)
        def _b():
            _gather(E)
            _pack_ab(b_out)

        def _pack_c(j):
            _gather(2 * E)
            vj = _bfly_max(jnp.where(iota == j, psort_v[...], -3.4e38))
            for c in range(DIM):
                c_scr[c, :] = buf_v[j, pl.ds(RANK * c, RANK)] * vj
            pltpu.sync_copy(c_scr, c_out.at[j])

        @pl.when(wid == 2)
        def _c0():
            _pack_c(0)

        @pl.when(wid == 3)
        def _c1():
            _pack_c(1)


def _apply_kernel(x_ref, s_ref, a_ref, b_ref, c_ref, wout_ref, v_ref, o_ref,
                  m_scr, wx_scr):
    @pl.when(pl.program_id(0) == 0)
    def _fold():
        wout = wout_ref[...]
        m_scr[0] = jnp.dot(wout, c_ref[0], preferred_element_type=jnp.float32,
                           precision=_APREC)
        m_scr[1] = jnp.dot(wout, c_ref[1], preferred_element_type=jnp.float32,
                           precision=_APREC)
        v = v_ref[...]
        col = jax.lax.broadcasted_iota(jnp.int32, (1, 16), 1)
        s = jnp.sum(jnp.where(col < 2, v, 0.0))
        wx_scr[...] = s * wout

    X = jnp.concatenate([x_ref[0, :, r, :] for r in range(ROWS_C)], axis=1)
    S = jnp.concatenate([s_ref[0, :, r, :] for r in range(ROWS_C)], axis=1)
    a = jnp.dot(a_ref[...], X, preferred_element_type=jnp.float32,
                precision=_APREC)
    b = jnp.dot(b_ref[...], S, preferred_element_type=jnp.float32,
                precision=_APREC)
    y = a * (b * jax.nn.sigmoid(b))
    o2 = (jnp.dot(m_scr[0], y[:RANK], preferred_element_type=jnp.float32,
                  precision=_APREC)
          + jnp.dot(m_scr[1], y[RANK:], preferred_element_type=jnp.float32,
                    precision=_APREC)
          + jnp.dot(wx_scr[...], X, preferred_element_type=jnp.float32,
                    precision=_APREC))
    for r in range(ROWS_C):
        o_ref[0, :, r, :] = o2[:, W * r:W * (r + 1)]


def _route(logits16, pw):
    return pl.kernel(
        _route_body,
        [
            jax.ShapeDtypeStruct((2 * RANK, DIM), jnp.float32),
            jax.ShapeDtypeStruct((2 * RANK, DIM), jnp.float32),
            jax.ShapeDtypeStruct((2, DIM, RANK), jnp.float32),
            jax.ShapeDtypeStruct((1, 16), jnp.float32),
        ],
        mesh=plsc.VectorSubcoreMesh(core_axis_name="c", subcore_axis_name="s"),
        compiler_params=pltpu.CompilerParams(needs_layout_passes=False),
        scratch_types=[
            pltpu.VMEM((1, 16), jnp.float32),
            pltpu.VMEM((16,), jnp.float32),
            pltpu.VMEM((16,), jnp.float32),
            pltpu.VMEM((16,), jnp.int32),
            pltpu.VMEM((E, RANK * DIM), jnp.float32),
            pltpu.VMEM((2 * RANK, DIM), jnp.float32),
            pltpu.VMEM((DIM, RANK), jnp.float32),
            pltpu.VMEM((1, 16), jnp.float32),
            pltpu.SemaphoreType.DMA,
        ],
    )(logits16, pw)


@jax.jit
def kernel(x, shared, W_mlp1, b_mlp1, W_mlp2, b_mlp2, Wg, Wf, P0, P1, P2, Wout):
    full = lambda s: pl.BlockSpec(s, lambda i: tuple(0 for _ in s))
    logits16, pw = pl.pallas_call(
        _stats_kernel,
        grid=(NA,),
        in_specs=[
            pl.BlockSpec((1, DIM, ROWS_A, W), lambda i: (0, 0, i, 0)),
            full((2 * DIM, DIM)), full((1, 2 * DIM)),
            full((DIM, 2 * DIM)), full((1, DIM)),
            full((E, DIM)), full((E, DIM)),
            full((E, RANK, DIM)), full((E, RANK, DIM)), full((E, DIM, RANK)),
        ],
        out_specs=[
            pl.BlockSpec((1, 16), lambda i: (0, 0)),
            pl.BlockSpec((3 * E, RANK * DIM), lambda i: (0, 0)),
        ],
        out_shape=[
            jax.ShapeDtypeStruct((1, 16), jnp.float32),
            jax.ShapeDtypeStruct((3 * E, RANK * DIM), jnp.float32),
        ],
        scratch_shapes=[
            pltpu.VMEM((DIM, 2, W), jnp.float32),
            pltpu.VMEM((DIM, 1, W), jnp.float32),
            pltpu.VMEM((2, DIM), jnp.float32),
        ],
    )(x, W_mlp1, b_mlp1.reshape(1, -1), W_mlp2, b_mlp2.reshape(1, -1),
      Wg, Wf, P0, P1, P2)

    a2, b2, c2, vs = _route(logits16, pw)

    out = pl.pallas_call(
        _apply_kernel,
        grid=(NC,),
        in_specs=[
            pl.BlockSpec((1, DIM, ROWS_C, W), lambda i: (0, 0, i, 0)),
            pl.BlockSpec((1, DIM, ROWS_C, W), lambda i: (0, 0, i, 0)),
            full((2 * RANK, DIM)), full((2 * RANK, DIM)),
            full((2, DIM, RANK)), full((DIM, DIM)), full((1, 16)),
        ],
        out_specs=pl.BlockSpec((1, DIM, ROWS_C, W), lambda i: (0, 0, i, 0)),
        out_shape=jax.ShapeDtypeStruct((1, DIM, H, W), jnp.float32),
        scratch_shapes=[
            pltpu.VMEM((2, DIM, RANK), jnp.float32),
            pltpu.VMEM((DIM, DIM), jnp.float32),
        ],
        compiler_params=pltpu.CompilerParams(
            vmem_limit_bytes=100 * 1024 * 1024),
    )(x, shared, a2, b2, c2, Wout, vs)
    return out
